# baseline (speedup 1.0000x reference)
import functools

import jax
import jax.numpy as jnp
from jax import lax
from jax.experimental import pallas as pl
from jax.experimental.pallas import tpu as pltpu
from jax.experimental.pallas import tpu_sc as plsc

_NUM_CLASSES = 1000
_PAD = 1024
_BATCH = 16384
_TAU = 0.7

_NC, _NS = 2, 16
_NW = _NC * _NS
_PER_W = _BATCH // _NW
_CHUNK = 32
_NCHUNK = _PER_W // _CHUNK
_NK = _PAD // 128
_NBUF = 3


def _softmax_body(t_ref, out_ref):
    x = t_ref[...] * (1.0 / _TAU)
    m = jnp.max(x, axis=-1, keepdims=True)
    e = jnp.exp(x - m)
    sm = e / jnp.sum(e, axis=-1, keepdims=True)
    out_ref[...] = jnp.pad(sm, ((0, 0), (0, _PAD - _NUM_CLASSES)))


def _row_softmax(t):
    return pl.pallas_call(
        _softmax_body,
        out_shape=jax.ShapeDtypeStruct((_NUM_CLASSES, _PAD), jnp.float32),
    )(t)


_mesh = plsc.VectorSubcoreMesh(
    core_axis_name="c", subcore_axis_name="s", num_cores=_NC, num_subcores=_NS
)


@functools.partial(
    pl.kernel,
    out_type=(
        jax.ShapeDtypeStruct((_BATCH, _NUM_CLASSES), jnp.float32),
        jax.ShapeDtypeStruct((_BATCH, 128), jnp.float32),
    ),
    mesh=_mesh,
    scratch_types=[
        pltpu.VMEM((_PER_W,), jnp.int32),
        *[pltpu.VMEM((_CHUNK, _PAD), jnp.float32) for _ in range(_NBUF)],
        *[pltpu.SemaphoreType.DMA for _ in range(2 * _NBUF)],
    ],
    compiler_params=pltpu.CompilerParams(use_tc_tiling_on_sc=True),
)
def _gather(table_hbm, idx_hbm, out_hbm, tail_hbm, idx_v, *bufsem):
    bufs = bufsem[:_NBUF]
    gsem = bufsem[_NBUF:2 * _NBUF]
    ssem = bufsem[2 * _NBUF:]
    wid = lax.axis_index("s") * _NC + lax.axis_index("c")
    base = wid * _PER_W
    pltpu.sync_copy(idx_hbm.at[pl.ds(base, _PER_W)], idx_v)

    def start_gather(j):
        b = j % _NBUF
        return pltpu.async_copy(
            table_hbm.at[idx_v.at[pl.ds(j * _CHUNK, _CHUNK)]], bufs[b], gsem[b]
        )

    def scatter_chunk(j):
        buf, sem = bufs[j % _NBUF], ssem[j % _NBUF]
        row0 = base + j * _CHUNK
        ds = []
        for k in range(_NK - 1):
            ds.append(
                pltpu.async_copy(
                    buf.at[:, pl.ds(128 * k, 128)],
                    out_hbm.at[pl.ds(row0, _CHUNK), pl.ds(128 * k, 128)],
                    sem,
                )
            )
        ds.append(
            pltpu.async_copy(
                buf.at[:, pl.ds(128 * (_NK - 1), 128)],
                tail_hbm.at[pl.ds(row0, _CHUNK)],
                sem,
            )
        )
        return ds

    gd = [None] * _NCHUNK
    sd = [None] * _NCHUNK
    for j in range(_NBUF - 1):
        gd[j] = start_gather(j)
    for j in range(_NCHUNK):
        jn = j + _NBUF - 1
        if jn < _NCHUNK:
            if jn >= _NBUF:
                for d in sd[jn - _NBUF]:
                    d.wait()
            gd[jn] = start_gather(jn)
        gd[j].wait()
        sd[j] = scatter_chunk(j)
    for j in range(_NCHUNK - _NBUF, _NCHUNK):
        for d in sd[j]:
            d.wait()


def _merge_body(main_ref, tail_ref, out_ref):
    out_ref[...] = tail_ref[...]


def _merge_tail(main, tail):
    nblk = 4
    rows = _BATCH // nblk
    return pl.pallas_call(
        _merge_body,
        grid=(nblk,),
        in_specs=[
            pl.BlockSpec((8, 128), lambda i: (0, 0)),
            pl.BlockSpec((rows, 128), lambda i: (i, 0)),
        ],
        out_specs=pl.BlockSpec((rows, 128), lambda i: (i, _NK - 1)),
        out_shape=jax.ShapeDtypeStruct((_BATCH, _NUM_CLASSES), jnp.float32),
        input_output_aliases={0: 0},
    )(main, tail)


def kernel(y, domain_id, T):
    t = lax.dynamic_index_in_dim(T, domain_id, axis=0, keepdims=False)
    t_j = _row_softmax(t)
    main, tail = _gather(t_j, y)
    return jnp.concatenate(
        [lax.slice(main, (0, 0), (_BATCH, 896)),
         lax.slice(tail, (0, 0), (_BATCH, _NUM_CLASSES - 896))], axis=1)


# speedup vs baseline: 1.4122x; 1.4122x over previous
import functools

import jax
import jax.numpy as jnp
from jax import lax
from jax.experimental import pallas as pl
from jax.experimental.pallas import tpu as pltpu
from jax.experimental.pallas import tpu_sc as plsc

_NUM_CLASSES = 1000
_PAD = 1024
_BATCH = 16384
_TAU = 0.7

_NC, _NS = 2, 16
_NW = _NC * _NS
_PER_W = _BATCH // _NW
_CHUNK = 32
_NCHUNK = _PER_W // _CHUNK
_NK = _PAD // 128
_NBUF = 3


def _softmax_body(t_ref, out_ref):
    x = t_ref[...] * (1.0 / _TAU)
    m = jnp.max(x, axis=-1, keepdims=True)
    e = jnp.exp(x - m)
    sm = e / jnp.sum(e, axis=-1, keepdims=True)
    out_ref[...] = jnp.pad(sm, ((0, 0), (0, _PAD - _NUM_CLASSES)))


def _row_softmax(t):
    return pl.pallas_call(
        _softmax_body,
        out_shape=jax.ShapeDtypeStruct((_NUM_CLASSES, _PAD), jnp.float32),
    )(t)


_mesh = plsc.VectorSubcoreMesh(
    core_axis_name="c", subcore_axis_name="s", num_cores=_NC, num_subcores=_NS
)


@functools.partial(
    pl.kernel,
    out_type=(
        jax.ShapeDtypeStruct((_BATCH, _NUM_CLASSES), jnp.float32),
        jax.ShapeDtypeStruct((_BATCH, 128), jnp.float32),
    ),
    mesh=_mesh,
    scratch_types=[
        pltpu.VMEM((_PER_W,), jnp.int32),
        *[pltpu.VMEM((_CHUNK, _PAD), jnp.float32) for _ in range(_NBUF)],
        *[pltpu.SemaphoreType.DMA for _ in range(2 * _NBUF)],
    ],
    compiler_params=pltpu.CompilerParams(use_tc_tiling_on_sc=True),
)
def _gather(table_hbm, idx_hbm, out_hbm, tail_hbm, idx_v, *bufsem):
    bufs = bufsem[:_NBUF]
    gsem = bufsem[_NBUF:2 * _NBUF]
    ssem = bufsem[2 * _NBUF:]
    wid = lax.axis_index("s") * _NC + lax.axis_index("c")
    base = wid * _PER_W
    pltpu.sync_copy(idx_hbm.at[pl.ds(base, _PER_W)], idx_v)

    def start_gather(j):
        b = j % _NBUF
        return pltpu.async_copy(
            table_hbm.at[idx_v.at[pl.ds(j * _CHUNK, _CHUNK)]], bufs[b], gsem[b]
        )

    def scatter_chunk(j):
        buf, sem = bufs[j % _NBUF], ssem[j % _NBUF]
        row0 = base + j * _CHUNK
        ds = []
        for k in range(_NK - 1):
            ds.append(
                pltpu.async_copy(
                    buf.at[:, pl.ds(128 * k, 128)],
                    out_hbm.at[pl.ds(row0, _CHUNK), pl.ds(128 * k, 128)],
                    sem,
                )
            )
        ds.append(
            pltpu.async_copy(
                buf.at[:, pl.ds(128 * (_NK - 1), 128)],
                tail_hbm.at[pl.ds(row0, _CHUNK)],
                sem,
            )
        )
        return ds

    gd = [None] * _NCHUNK
    sd = [None] * _NCHUNK
    for j in range(_NBUF - 1):
        gd[j] = start_gather(j)
    for j in range(_NCHUNK):
        jn = j + _NBUF - 1
        if jn < _NCHUNK:
            if jn >= _NBUF:
                for d in sd[jn - _NBUF]:
                    d.wait()
            gd[jn] = start_gather(jn)
        gd[j].wait()
        sd[j] = scatter_chunk(j)
    for j in range(_NCHUNK - _NBUF, _NCHUNK):
        for d in sd[j]:
            d.wait()


def _merge_body(main_ref, tail_ref, out_ref):
    out_ref[...] = tail_ref[...]


def _merge_tail(main, tail):
    nblk = 4
    rows = _BATCH // nblk
    return pl.pallas_call(
        _merge_body,
        grid=(nblk,),
        in_specs=[
            pl.BlockSpec((8, 128), lambda i: (0, 0)),
            pl.BlockSpec((rows, 128), lambda i: (i, 0)),
        ],
        out_specs=pl.BlockSpec((rows, 128), lambda i: (i, _NK - 1)),
        out_shape=jax.ShapeDtypeStruct((_BATCH, _NUM_CLASSES), jnp.float32),
        input_output_aliases={0: 0},
    )(main, tail)


def kernel(y, domain_id, T):
    t = lax.dynamic_index_in_dim(T, domain_id, axis=0, keepdims=False)
    t_j = _row_softmax(t)
    main, tail = _gather(t_j, y)
    return _merge_tail(main, tail)
